# Initial kernel scaffold; baseline (speedup 1.0000x reference)
import jax
import jax.numpy as jnp
from jax.experimental import pallas as pl


def kernel(epoch, v1, v2, y, idx, memory_v1, memory_v2):
    raise NotImplementedError("write your pallas kernel here")


# speedup vs baseline: 4.0576x; 4.0576x over previous
import functools

import jax
import jax.numpy as jnp
from jax import lax
from jax.experimental import pallas as pl
from jax.experimental.pallas import tpu as pltpu
from jax.experimental.pallas import tpu_sc as plsc

_P = 50
_K = 1024
_P2 = 10
_T = 0.07
_MOM = 0.5
_D = 64
_OUT_ROWS = 1000000
_B = 1024
_KP = _K + _P
_KPAD = 1088
_NW = 32
_BPW = _B // _NW
_CH = 272

_mesh = plsc.VectorSubcoreMesh(
    core_axis_name="c", subcore_axis_name="s", num_cores=2, num_subcores=16
)


def _load_vregs(vvec_ref):
  return [vvec_ref[pl.ds(16 * j, 16)] for j in range(_D // 16)]


def _dot_group(rows_ref, rbase, vregs, iota16):
  acc = jnp.zeros((16,), jnp.float32)
  ridx = iota16 + rbase
  for d in range(_D):
    col = plsc.load_gather(rows_ref, [ridx, jnp.full((16,), d, jnp.int32)])
    acc = acc + col * vregs[d // 16][d % 16]
  return acc


def _dot_ss_group(rows_ref, rbase, vregs, iota16):
  accd = jnp.zeros((16,), jnp.float32)
  accs = jnp.zeros((16,), jnp.float32)
  ridx = iota16 + rbase
  for d in range(_D):
    col = plsc.load_gather(rows_ref, [ridx, jnp.full((16,), d, jnp.int32)])
    accd = accd + col * vregs[d // 16][d % 16]
    accs = accs + col * col
  return accd, accs


@functools.partial(
    pl.kernel,
    mesh=_mesh,
    compiler_params=pltpu.CompilerParams(
        needs_layout_passes=False, use_tc_tiling_on_sc=False
    ),
    out_type=[
        jax.ShapeDtypeStruct((_B, _KPAD), jnp.float32),
        jax.ShapeDtypeStruct((_B, _D), jnp.float32),
        jax.ShapeDtypeStruct((_B, _D), jnp.float32),
        jax.ShapeDtypeStruct((_B, _D), jnp.float32),
        jax.ShapeDtypeStruct((_B, _D), jnp.float32),
        jax.ShapeDtypeStruct((_B, _D), jnp.float32),
    ],
    scratch_types=[
        pltpu.VMEM((_KPAD,), jnp.int32),
        pltpu.VMEM((_CH, _D), jnp.float32),
        pltpu.VMEM((_D, _D), jnp.float32),
        pltpu.VMEM((_D, _D), jnp.float32),
        pltpu.VMEM((_KPAD,), jnp.float32),
        pltpu.VMEM((_D,), jnp.float32),
        pltpu.VMEM((_D,), jnp.float32),
        pltpu.VMEM((_D,), jnp.float32),
        pltpu.VMEM((_D,), jnp.float32),
        pltpu.VMEM((_D,), jnp.float32),
        pltpu.VMEM((_D,), jnp.float32),
        pltpu.VMEM((_BPW,), jnp.int32),
        pltpu.VMEM((_BPW, _D), jnp.float32),
        pltpu.SemaphoreType.DMA,
    ],
)
def _gather_dot(m1_hbm, m2_hbm, idx_hbm, v1_hbm, v2_hbm, y_hbm,
                dots_hbm, dv1_hbm, ss1_hbm, d2_hbm, ss2_hbm, ry_hbm,
                idx_v, rows_v, p1_v, p2_v, dots_v, v1_v, v2_v,
                pdv1_v, pss1_v, pd2_v, pss2_v, y_v, yrows_v, sem):
  wid = lax.axis_index("s") * 2 + lax.axis_index("c")
  base_b = wid * _BPW
  iota16 = lax.iota(jnp.int32, 16)

  pltpu.sync_copy(y_hbm.at[pl.ds(base_b, _BPW)], y_v)
  pltpu.async_copy(m1_hbm.at[y_v], yrows_v, sem).wait()
  pltpu.sync_copy(yrows_v, ry_hbm.at[pl.ds(base_b, _BPW)])

  def per_b(i, carry):
    b = base_b + i
    pltpu.sync_copy(idx_hbm.at[b], idx_v)
    pltpu.sync_copy(v1_hbm.at[b], v1_v)
    pltpu.sync_copy(v2_hbm.at[b], v2_v)

    pltpu.async_copy(m1_hbm.at[idx_v.at[pl.ds(0, _D)]], p1_v, sem).wait()
    pltpu.async_copy(m2_hbm.at[idx_v.at[pl.ds(0, _D)]], p2_v, sem).wait()
    v1_regs = _load_vregs(v1_v)
    v2_regs = _load_vregs(v2_v)
    for g in range(4):
      d1, s1 = _dot_ss_group(p1_v, g * 16, v1_regs, iota16)
      pdv1_v[pl.ds(g * 16, 16)] = d1
      pss1_v[pl.ds(g * 16, 16)] = s1
      d2v, s2v = _dot_ss_group(p2_v, g * 16, v2_regs, iota16)
      pd2_v[pl.ds(g * 16, 16)] = d2v
      pss2_v[pl.ds(g * 16, 16)] = s2v
    pltpu.sync_copy(pdv1_v, dv1_hbm.at[b])
    pltpu.sync_copy(pss1_v, ss1_hbm.at[b])
    pltpu.sync_copy(pd2_v, d2_hbm.at[b])
    pltpu.sync_copy(pss2_v, ss2_hbm.at[b])

    def chunk(c, carry2):
      pltpu.async_copy(
          m1_hbm.at[idx_v.at[pl.ds(c * _CH, _CH)]], rows_v, sem
      ).wait()

      def group(g, carry3):
        acc = _dot_group(rows_v, g * 16, v2_regs, iota16)
        dots_v[pl.ds(c * _CH + g * 16, 16)] = acc
        return carry3

      lax.fori_loop(0, _CH // 16, group, 0)
      return carry2

    lax.fori_loop(0, _KPAD // _CH, chunk, 0)
    pltpu.sync_copy(dots_v, dots_hbm.at[b])
    return carry

  lax.fori_loop(0, _BPW, per_b, 0)


def _epilogue_body(dots_ref, dv1_ref, ss1_ref, d2_ref, ss2_ref,
                   v1_ref, v2_ref, ry_ref, out_ref, lpos_ref):
  v1 = v1_ref[...]
  v2 = v2_ref[...]
  ex = jnp.exp(dots_ref[...] * (1.0 / _T))

  n1 = jnp.sum(v1 * v1, axis=1, keepdims=True)
  n2 = jnp.sum(v2 * v2, axis=1, keepdims=True)
  t = dv1_ref[...][:, :_P] / jnp.sqrt(ss1_ref[...][:, :_P] * n1)
  s = d2_ref[...][:, :_P] / jnp.sqrt(ss2_ref[...][:, :_P] * n2)
  diff = t - s

  iota_p = lax.broadcasted_iota(jnp.int32, (_B, _P), 1)
  onehots = []
  for j in range(_P2):
    m = jnp.max(diff, axis=1, keepdims=True)
    amax = jnp.min(jnp.where(diff >= m, iota_p, _P), axis=1, keepdims=True)
    sel = jnp.zeros_like(amax) if j == 0 else amax
    onehots.append((iota_p == sel).astype(jnp.float32))
    diff = jnp.where(iota_p == amax, -jnp.inf, diff)
  oh = jnp.stack(onehots, axis=1)

  pos = jnp.sum(oh * ex[:, None, :_P], axis=2)
  neg = ex[:, _P:_KP]
  z = (jnp.sum(pos) + jnp.sum(neg)) / (_B * (_P2 + _K)) * _OUT_ROWS
  inv_z = 1.0 / z
  out_ref[:, :_P2] = pos * inv_z
  out_ref[:, _P2:] = neg * inv_z

  lp = ry_ref[...] * _MOM + v1 * (1.0 - _MOM)
  lp = lp / jnp.sqrt(jnp.sum(lp * lp, axis=1, keepdims=True))
  lpos_ref[...] = lp


_epilogue = pl.pallas_call(
    _epilogue_body,
    out_shape=[
        jax.ShapeDtypeStruct((_B, _P2 + _K), jnp.float32),
        jax.ShapeDtypeStruct((_B, _D), jnp.float32),
    ],
)

_COPY_ROWS = 10000


def _copy_body(src_ref, dst_ref):
  dst_ref[...] = src_ref[...]


_copy = pl.pallas_call(
    _copy_body,
    grid=(_OUT_ROWS // _COPY_ROWS,),
    in_specs=[pl.BlockSpec((_COPY_ROWS, _D), lambda i: (i, 0))],
    out_specs=pl.BlockSpec((_COPY_ROWS, _D), lambda i: (i, 0)),
    out_shape=jax.ShapeDtypeStruct((_OUT_ROWS, _D), jnp.float32),
)


@functools.partial(
    pl.kernel,
    mesh=_mesh,
    compiler_params=pltpu.CompilerParams(
        needs_layout_passes=False, use_tc_tiling_on_sc=False
    ),
    out_type=(),
    scratch_types=[
        pltpu.VMEM((_BPW,), jnp.int32),
        pltpu.VMEM((_BPW, _D), jnp.float32),
        pltpu.SemaphoreType.DMA,
    ],
)
def _scatter(y_hbm, lpos_hbm, mem_hbm, y_v, rows_v, sem):
  wid = lax.axis_index("s") * 2 + lax.axis_index("c")
  base = wid * _BPW
  pltpu.sync_copy(y_hbm.at[pl.ds(base, _BPW)], y_v)
  pltpu.sync_copy(lpos_hbm.at[pl.ds(base, _BPW)], rows_v)
  pltpu.async_copy(rows_v, mem_hbm.at[y_v], sem).wait()


def kernel(epoch, v1, v2, y, idx, memory_v1, memory_v2):
  del epoch
  idx_pad = jnp.pad(idx.astype(jnp.int32), ((0, 0), (0, _KPAD - _KP)))
  y32 = y.astype(jnp.int32)

  dots, dv1, ss1, d2, ss2, ry = _gather_dot(
      memory_v1, memory_v2, idx_pad, v1, v2, y32
  )
  out2d, lpos = _epilogue(dots, dv1, ss1, d2, ss2, v1, v2, ry)

  copied = _copy(memory_v1)
  mem_ref = jax.new_ref(copied)
  _scatter(y32, lpos, mem_ref)
  mem_new = mem_ref[...]

  return out2d[..., None], mem_new


# speedup vs baseline: 4.9352x; 1.2163x over previous
import functools

import jax
import jax.numpy as jnp
from jax import lax
from jax.experimental import pallas as pl
from jax.experimental.pallas import tpu as pltpu
from jax.experimental.pallas import tpu_sc as plsc

_P = 50
_K = 1024
_P2 = 10
_T = 0.07
_MOM = 0.5
_D = 64
_OUT_ROWS = 1000000
_B = 1024
_KP = _K + _P
_KPAD = 1088
_NW = 32
_BPW = _B // _NW
_CH = 272

_mesh = plsc.VectorSubcoreMesh(
    core_axis_name="c", subcore_axis_name="s", num_cores=2, num_subcores=16
)
_sc_params = pltpu.CompilerParams(
    needs_layout_passes=False, use_tc_tiling_on_sc=False
)


def _vregs(ref2d, i):
  return [ref2d[i, pl.ds(16 * j, 16)] for j in range(_D // 16)]


def _dot_group(rows_ref, rbase, vregs, iota16):
  acc = jnp.zeros((16,), jnp.float32)
  ridx = iota16 + rbase
  for d in range(_D):
    col = plsc.load_gather(rows_ref, [ridx, jnp.full((16,), d, jnp.int32)])
    acc = acc + col * vregs[d // 16][d % 16]
  return acc


def _dot_ss_group(rows_ref, rbase, vregs, iota16):
  accd = jnp.zeros((16,), jnp.float32)
  accs = jnp.zeros((16,), jnp.float32)
  ridx = iota16 + rbase
  for d in range(_D):
    col = plsc.load_gather(rows_ref, [ridx, jnp.full((16,), d, jnp.int32)])
    accd = accd + col * vregs[d // 16][d % 16]
    accs = accs + col * col
  return accd, accs


@functools.partial(
    pl.kernel,
    mesh=_mesh,
    compiler_params=_sc_params,
    out_type=[
        jax.ShapeDtypeStruct((_B, _KPAD), jnp.float32),
        jax.ShapeDtypeStruct((_B, 4 * _D), jnp.float32),
        jax.ShapeDtypeStruct((_B, _D), jnp.float32),
    ],
    scratch_types=[
        pltpu.VMEM((_BPW, _KPAD), jnp.int32),
        pltpu.VMEM((_CH, _D), jnp.float32),
        pltpu.VMEM((_CH, _D), jnp.float32),
        pltpu.VMEM((_D, _D), jnp.float32),
        pltpu.VMEM((_D, _D), jnp.float32),
        pltpu.VMEM((_KPAD,), jnp.float32),
        pltpu.VMEM((4 * _D,), jnp.float32),
        pltpu.VMEM((_BPW, _D), jnp.float32),
        pltpu.VMEM((_BPW, _D), jnp.float32),
        pltpu.VMEM((_BPW,), jnp.int32),
        pltpu.VMEM((_BPW, _D), jnp.float32),
        pltpu.SemaphoreType.DMA,
        pltpu.SemaphoreType.DMA,
        pltpu.SemaphoreType.DMA,
        pltpu.SemaphoreType.DMA,
    ],
)
def _gather_dot(m1_hbm, m2_hbm, idx_hbm, v1_hbm, v2_hbm, y_hbm,
                dots_hbm, stats_hbm, ry_hbm,
                idx_v, buf0_v, buf1_v, p1_v, p2_v, dots_v, stats_v,
                v1s_v, v2s_v, y_v, yrows_v, sem0, sem1, semp1, semp2):
  wid = lax.axis_index("s") * 2 + lax.axis_index("c")
  base_b = wid * _BPW
  iota16 = lax.iota(jnp.int32, 16)

  pltpu.sync_copy(idx_hbm.at[pl.ds(base_b, _BPW)], idx_v)
  pltpu.sync_copy(v1_hbm.at[pl.ds(base_b, _BPW)], v1s_v)
  pltpu.sync_copy(v2_hbm.at[pl.ds(base_b, _BPW)], v2s_v)

  pltpu.sync_copy(y_hbm.at[pl.ds(base_b, _BPW)], y_v)
  pltpu.async_copy(m1_hbm.at[y_v], yrows_v, semp1).wait()
  pltpu.sync_copy(yrows_v, ry_hbm.at[pl.ds(base_b, _BPW)])

  def per_b(i, carry):
    b = base_b + i
    v1_regs = _vregs(v1s_v, i)
    v2_regs = _vregs(v2s_v, i)

    cp_p1 = pltpu.async_copy(m1_hbm.at[idx_v.at[i, pl.ds(0, _D)]], p1_v, semp1)
    cp_p2 = pltpu.async_copy(m2_hbm.at[idx_v.at[i, pl.ds(0, _D)]], p2_v, semp2)
    bufs = (buf0_v, buf1_v)
    sems = (sem0, sem1)
    cps = [
        pltpu.async_copy(
            m1_hbm.at[idx_v.at[i, pl.ds(c * _CH, _CH)]], bufs[c % 2], sems[c % 2]
        )
        for c in range(2)
    ]

    cp_p1.wait()

    def pgroup1(g, c3):
      d1, s1 = _dot_ss_group(p1_v, g * 16, v1_regs, iota16)
      stats_v[pl.ds(g * 16, 16)] = d1
      stats_v[pl.ds(_D + g * 16, 16)] = s1
      return c3

    lax.fori_loop(0, 4, pgroup1, 0)
    cp_p2.wait()

    def pgroup2(g, c3):
      d2, s2 = _dot_ss_group(p2_v, g * 16, v2_regs, iota16)
      stats_v[pl.ds(2 * _D + g * 16, 16)] = d2
      stats_v[pl.ds(3 * _D + g * 16, 16)] = s2
      return c3

    lax.fori_loop(0, 4, pgroup2, 0)

    for c in range(_KPAD // _CH):
      cps[c].wait()
      buf = bufs[c % 2]

      def group(g, c3, c=c, buf=buf):
        acc = _dot_group(buf, g * 16, v2_regs, iota16)
        dots_v[pl.ds(c * _CH + g * 16, 16)] = acc
        return c3

      lax.fori_loop(0, _CH // 16, group, 0)
      if c + 2 < _KPAD // _CH:
        cps.append(
            pltpu.async_copy(
                m1_hbm.at[idx_v.at[i, pl.ds((c + 2) * _CH, _CH)]],
                bufs[c % 2], sems[c % 2],
            )
        )

    pltpu.sync_copy(dots_v, dots_hbm.at[b])
    pltpu.sync_copy(stats_v, stats_hbm.at[b])
    return carry

  lax.fori_loop(0, _BPW, per_b, 0)


def _epilogue_body(dots_ref, stats_ref, v1_ref, v2_ref, ry_ref,
                   out_ref, lpos_ref):
  v1 = v1_ref[...]
  v2 = v2_ref[...]
  ex = jnp.exp(dots_ref[...] * (1.0 / _T))

  stats = stats_ref[...]
  n1 = jnp.sum(v1 * v1, axis=1, keepdims=True)
  n2 = jnp.sum(v2 * v2, axis=1, keepdims=True)
  t = stats[:, :_P] / jnp.sqrt(stats[:, _D:_D + _P] * n1)
  s = stats[:, 2 * _D:2 * _D + _P] / jnp.sqrt(stats[:, 3 * _D:3 * _D + _P] * n2)
  diff = t - s

  iota_p = lax.broadcasted_iota(jnp.int32, (_B, _P), 1)
  onehots = []
  for j in range(_P2):
    m = jnp.max(diff, axis=1, keepdims=True)
    amax = jnp.min(jnp.where(diff >= m, iota_p, _P), axis=1, keepdims=True)
    sel = jnp.zeros_like(amax) if j == 0 else amax
    onehots.append((iota_p == sel).astype(jnp.float32))
    diff = jnp.where(iota_p == amax, -jnp.inf, diff)
  oh = jnp.stack(onehots, axis=1)

  pos = jnp.sum(oh * ex[:, None, :_P], axis=2)
  neg = ex[:, _P:_KP]
  z = (jnp.sum(pos) + jnp.sum(neg)) / (_B * (_P2 + _K)) * _OUT_ROWS
  inv_z = 1.0 / z
  out_ref[:, :_P2] = pos * inv_z
  out_ref[:, _P2:] = neg * inv_z

  lp = ry_ref[...] * _MOM + v1 * (1.0 - _MOM)
  lp = lp / jnp.sqrt(jnp.sum(lp * lp, axis=1, keepdims=True))
  lpos_ref[...] = lp


_epilogue = pl.pallas_call(
    _epilogue_body,
    out_shape=[
        jax.ShapeDtypeStruct((_B, _P2 + _K), jnp.float32),
        jax.ShapeDtypeStruct((_B, _D), jnp.float32),
    ],
)


@functools.partial(
    pl.kernel,
    mesh=_mesh,
    compiler_params=_sc_params,
    out_type=(),
    scratch_types=[
        pltpu.VMEM((_BPW,), jnp.int32),
        pltpu.VMEM((_BPW, _D), jnp.float32),
        pltpu.SemaphoreType.DMA,
    ],
)
def _scatter(y_hbm, lpos_hbm, mem_hbm, y_v, rows_v, sem):
  wid = lax.axis_index("s") * 2 + lax.axis_index("c")
  base = wid * _BPW
  pltpu.sync_copy(y_hbm.at[pl.ds(base, _BPW)], y_v)
  pltpu.sync_copy(lpos_hbm.at[pl.ds(base, _BPW)], rows_v)
  pltpu.async_copy(rows_v, mem_hbm.at[y_v], sem).wait()


def kernel(epoch, v1, v2, y, idx, memory_v1, memory_v2):
  del epoch
  idx_pad = jnp.pad(idx.astype(jnp.int32), ((0, 0), (0, _KPAD - _KP)))
  y32 = y.astype(jnp.int32)

  mem_ref = jax.new_ref(memory_v1)

  dots, stats, ry = _gather_dot(mem_ref, memory_v2, idx_pad, v1, v2, y32)
  out2d, lpos = _epilogue(dots, stats, v1, v2, ry)

  _scatter(y32, lpos, mem_ref)
  mem_new = mem_ref[...]

  return out2d[..., None], mem_new


# speedup vs baseline: 6.8190x; 1.3817x over previous
import functools

import jax
import jax.numpy as jnp
from jax import lax
from jax.experimental import pallas as pl
from jax.experimental.pallas import tpu as pltpu
from jax.experimental.pallas import tpu_sc as plsc

_P = 50
_K = 1024
_P2 = 10
_T = 0.07
_MOM = 0.5
_D = 64
_OUT_ROWS = 1000000
_B = 1024
_KP = _K + _P
_KPAD = 1088
_NW = 32
_BPW = _B // _NW
_CH = 272

_mesh = plsc.VectorSubcoreMesh(
    core_axis_name="c", subcore_axis_name="s", num_cores=2, num_subcores=16
)
_sc_params = pltpu.CompilerParams(
    needs_layout_passes=False, use_tc_tiling_on_sc=False
)


def _fill_circ(circ_ref, src_ref, i):
  for j in range(_D // 16):
    v = src_ref[i, pl.ds(16 * j, 16)]
    circ_ref[pl.ds(16 * j, 16)] = v
    circ_ref[pl.ds(_D + 16 * j, 16)] = v


def _dot_group(rows_ref, rbase, circ_ref, iota16):
  acc = jnp.zeros((16,), jnp.float32)
  ridx = iota16 + rbase
  for d in range(_D):
    colidx = (iota16 + d) & (_D - 1)
    col = plsc.load_gather(rows_ref, [ridx, colidx])
    acc = acc + col * circ_ref[pl.ds(d, 16)]
  return acc


def _dot_ss_group(rows_ref, rbase, circ_ref, iota16):
  accd = jnp.zeros((16,), jnp.float32)
  accs = jnp.zeros((16,), jnp.float32)
  ridx = iota16 + rbase
  for d in range(_D):
    colidx = (iota16 + d) & (_D - 1)
    col = plsc.load_gather(rows_ref, [ridx, colidx])
    accd = accd + col * circ_ref[pl.ds(d, 16)]
    accs = accs + col * col
  return accd, accs


@functools.partial(
    pl.kernel,
    mesh=_mesh,
    compiler_params=_sc_params,
    out_type=jax.ShapeDtypeStruct((_B, _KPAD), jnp.float32),
    scratch_types=[
        pltpu.VMEM((_BPW, _KPAD), jnp.int32),
        pltpu.VMEM((_CH, _D), jnp.float32),
        pltpu.VMEM((_CH, _D), jnp.float32),
        pltpu.VMEM((_KPAD,), jnp.float32),
        pltpu.VMEM((2 * _D,), jnp.float32),
        pltpu.VMEM((_BPW, _D), jnp.float32),
        pltpu.SemaphoreType.DMA,
        pltpu.SemaphoreType.DMA,
    ],
)
def _main_dots(m1_hbm, idx_hbm, v2_hbm,
               dots_hbm,
               idx_v, buf0_v, buf1_v, dots_v, cv2_v, v2s_v, sem0, sem1):
  wid = lax.axis_index("s") * 2 + lax.axis_index("c")
  base_b = wid * _BPW
  iota16 = lax.iota(jnp.int32, 16)

  pltpu.sync_copy(idx_hbm.at[pl.ds(base_b, _BPW)], idx_v)
  pltpu.sync_copy(v2_hbm.at[pl.ds(base_b, _BPW)], v2s_v)

  def per_b(i, carry):
    b = base_b + i
    _fill_circ(cv2_v, v2s_v, i)

    bufs = (buf0_v, buf1_v)
    sems = (sem0, sem1)
    cps = [
        pltpu.async_copy(
            m1_hbm.at[idx_v.at[i, pl.ds(c * _CH, _CH)]], bufs[c % 2], sems[c % 2]
        )
        for c in range(2)
    ]

    for c in range(_KPAD // _CH):
      cps[c].wait()
      buf = bufs[c % 2]

      def group(g, c3, c=c, buf=buf):
        acc = _dot_group(buf, g * 16, cv2_v, iota16)
        dots_v[pl.ds(c * _CH + g * 16, 16)] = acc
        return c3

      lax.fori_loop(0, _CH // 16, group, 0)
      if c + 2 < _KPAD // _CH:
        cps.append(
            pltpu.async_copy(
                m1_hbm.at[idx_v.at[i, pl.ds((c + 2) * _CH, _CH)]],
                bufs[c % 2], sems[c % 2],
            )
        )

    pltpu.sync_copy(dots_v, dots_hbm.at[b])
    return carry

  lax.fori_loop(0, _BPW, per_b, 0)


@functools.partial(
    pl.kernel,
    mesh=_mesh,
    compiler_params=_sc_params,
    out_type=[
        jax.ShapeDtypeStruct((_B, 4 * _D), jnp.float32),
        jax.ShapeDtypeStruct((_B, _D), jnp.float32),
    ],
    scratch_types=[
        pltpu.VMEM((_BPW, _KPAD), jnp.int32),
        pltpu.VMEM((_D, _D), jnp.float32),
        pltpu.VMEM((_D, _D), jnp.float32),
        pltpu.VMEM((_D, _D), jnp.float32),
        pltpu.VMEM((_D, _D), jnp.float32),
        pltpu.VMEM((4 * _D,), jnp.float32),
        pltpu.VMEM((2 * _D,), jnp.float32),
        pltpu.VMEM((2 * _D,), jnp.float32),
        pltpu.VMEM((_BPW, _D), jnp.float32),
        pltpu.VMEM((_BPW, _D), jnp.float32),
        pltpu.VMEM((_BPW,), jnp.int32),
        pltpu.VMEM((_BPW, _D), jnp.float32),
        pltpu.SemaphoreType.DMA,
        pltpu.SemaphoreType.DMA,
        pltpu.SemaphoreType.DMA,
        pltpu.SemaphoreType.DMA,
    ],
)
def _pos_stats(m1_hbm, m2_hbm, idx_hbm, v1_hbm, v2_hbm, y_hbm,
               stats_hbm, ry_hbm,
               idx_v, p1a_v, p1b_v, p2a_v, p2b_v, stats_v,
               cv1_v, cv2_v, v1s_v, v2s_v, y_v, yrows_v,
               sem1a, sem1b, sem2a, sem2b):
  wid = lax.axis_index("s") * 2 + lax.axis_index("c")
  base_b = wid * _BPW
  iota16 = lax.iota(jnp.int32, 16)

  pltpu.sync_copy(idx_hbm.at[pl.ds(base_b, _BPW)], idx_v)
  pltpu.sync_copy(v1_hbm.at[pl.ds(base_b, _BPW)], v1s_v)
  pltpu.sync_copy(v2_hbm.at[pl.ds(base_b, _BPW)], v2s_v)

  pltpu.sync_copy(y_hbm.at[pl.ds(base_b, _BPW)], y_v)
  pltpu.async_copy(m1_hbm.at[y_v], yrows_v, sem1a).wait()
  pltpu.sync_copy(yrows_v, ry_hbm.at[pl.ds(base_b, _BPW)])

  def issue(i, p1buf, p2buf, sa, sb):
    pltpu.async_copy(m1_hbm.at[idx_v.at[i, pl.ds(0, _D)]], p1buf, sa)
    pltpu.async_copy(m2_hbm.at[idx_v.at[i, pl.ds(0, _D)]], p2buf, sb)

  def compute_b(i, p1buf, p2buf, sa, sb):
    b = base_b + i
    _fill_circ(cv1_v, v1s_v, i)
    _fill_circ(cv2_v, v2s_v, i)
    pltpu.make_async_copy(
        m1_hbm.at[idx_v.at[i, pl.ds(0, _D)]], p1buf, sa
    ).wait()

    def pgroup1(g, c3):
      d1, ss1 = _dot_ss_group(p1buf, g * 16, cv1_v, iota16)
      stats_v[pl.ds(g * 16, 16)] = d1
      stats_v[pl.ds(_D + g * 16, 16)] = ss1
      return c3

    lax.fori_loop(0, 4, pgroup1, 0)
    pltpu.make_async_copy(
        m2_hbm.at[idx_v.at[i, pl.ds(0, _D)]], p2buf, sb
    ).wait()

    def pgroup2(g, c3):
      d2, ss2 = _dot_ss_group(p2buf, g * 16, cv2_v, iota16)
      stats_v[pl.ds(2 * _D + g * 16, 16)] = d2
      stats_v[pl.ds(3 * _D + g * 16, 16)] = ss2
      return c3

    lax.fori_loop(0, 4, pgroup2, 0)
    pltpu.sync_copy(stats_v, stats_hbm.at[b])

  issue(0, p1a_v, p2a_v, sem1a, sem2a)
  issue(1, p1b_v, p2b_v, sem1b, sem2b)

  def pair(j, carry):
    i0 = 2 * j
    compute_b(i0, p1a_v, p2a_v, sem1a, sem2a)

    @pl.when(j < _BPW // 2 - 1)
    def _():
      issue(i0 + 2, p1a_v, p2a_v, sem1a, sem2a)

    compute_b(i0 + 1, p1b_v, p2b_v, sem1b, sem2b)

    @pl.when(j < _BPW // 2 - 1)
    def _():
      issue(i0 + 3, p1b_v, p2b_v, sem1b, sem2b)

    return carry

  lax.fori_loop(0, _BPW // 2, pair, 0)


def _epilogue_body(dots_ref, stats_ref, v1_ref, v2_ref, ry_ref,
                   out_ref, lpos_ref):
  v1 = v1_ref[...]
  v2 = v2_ref[...]
  ex = jnp.exp(dots_ref[...] * (1.0 / _T))

  stats = stats_ref[...]
  n1 = jnp.sum(v1 * v1, axis=1, keepdims=True)
  n2 = jnp.sum(v2 * v2, axis=1, keepdims=True)
  t = stats[:, :_P] / jnp.sqrt(stats[:, _D:_D + _P] * n1)
  s = stats[:, 2 * _D:2 * _D + _P] / jnp.sqrt(stats[:, 3 * _D:3 * _D + _P] * n2)
  diff = t - s

  iota_p = lax.broadcasted_iota(jnp.int32, (_B, _P), 1)
  onehots = []
  for j in range(_P2):
    m = jnp.max(diff, axis=1, keepdims=True)
    amax = jnp.min(jnp.where(diff >= m, iota_p, _P), axis=1, keepdims=True)
    sel = jnp.zeros_like(amax) if j == 0 else amax
    onehots.append((iota_p == sel).astype(jnp.float32))
    diff = jnp.where(iota_p == amax, -jnp.inf, diff)
  oh = jnp.stack(onehots, axis=1)

  pos = jnp.sum(oh * ex[:, None, :_P], axis=2)
  neg = ex[:, _P:_KP]
  z = (jnp.sum(pos) + jnp.sum(neg)) / (_B * (_P2 + _K)) * _OUT_ROWS
  inv_z = 1.0 / z
  out_ref[:, :_P2] = pos * inv_z
  out_ref[:, _P2:] = neg * inv_z

  lp = ry_ref[...] * _MOM + v1 * (1.0 - _MOM)
  lp = lp / jnp.sqrt(jnp.sum(lp * lp, axis=1, keepdims=True))
  lpos_ref[...] = lp


_epilogue = pl.pallas_call(
    _epilogue_body,
    out_shape=[
        jax.ShapeDtypeStruct((_B, _P2 + _K), jnp.float32),
        jax.ShapeDtypeStruct((_B, _D), jnp.float32),
    ],
)


@functools.partial(
    pl.kernel,
    mesh=_mesh,
    compiler_params=_sc_params,
    out_type=(),
    scratch_types=[
        pltpu.VMEM((_BPW,), jnp.int32),
        pltpu.VMEM((_BPW, _D), jnp.float32),
        pltpu.SemaphoreType.DMA,
    ],
)
def _scatter(y_hbm, lpos_hbm, mem_hbm, y_v, rows_v, sem):
  wid = lax.axis_index("s") * 2 + lax.axis_index("c")
  base = wid * _BPW
  pltpu.sync_copy(y_hbm.at[pl.ds(base, _BPW)], y_v)
  pltpu.sync_copy(lpos_hbm.at[pl.ds(base, _BPW)], rows_v)
  pltpu.async_copy(rows_v, mem_hbm.at[y_v], sem).wait()


def _linear(bank):
  flat = lax.optimization_barrier(jnp.reshape(bank, (_OUT_ROWS * _D,)))
  return jnp.reshape(flat, (_OUT_ROWS, _D))


def kernel(epoch, v1, v2, y, idx, memory_v1, memory_v2):
  del epoch
  idx_pad = jnp.pad(idx.astype(jnp.int32), ((0, 0), (0, _KPAD - _KP)))
  y32 = y.astype(jnp.int32)

  mem_ref = jax.new_ref(_linear(memory_v1))
  m2_lin = _linear(memory_v2)

  dots = _main_dots(mem_ref, idx_pad, v2)
  stats, ry = _pos_stats(mem_ref, m2_lin, idx_pad, v1, v2, y32)
  out2d, lpos = _epilogue(dots, stats, v1, v2, ry)

  _scatter(y32, lpos, mem_ref)
  mem_new = mem_ref[...]

  return out2d[..., None], mem_new
